# baseline (speedup 1.0000x reference)
import jax
import jax.numpy as jnp
from jax import lax
from jax.experimental import pallas as pl
from jax.experimental.pallas import tpu as pltpu
from jax.experimental.pallas import tpu_sc as plsc

NC = 2
NS = 16
NW = NC * NS
LANES = 16
K = 128


def _tc_pre(x, wcat, u2d, b1, N, D, R):
    def body(x_ref, wcat_ref, u_ref, b1_ref, h2_ref, a_ref, bb_ref):
        W = wcat_ref[:, :D]
        u = u_ref[...]
        v = jnp.dot(u, W, preferred_element_type=jnp.float32)
        v = v / (jnp.sqrt(jnp.sum(v * v)) + 1e-12)
        wv = lax.dot_general(v, W, (((1,), (1,)), ((), ())),
                             preferred_element_type=jnp.float32)
        u2 = wv / (jnp.sqrt(jnp.sum(wv * wv)) + 1e-12)
        sigma = jnp.abs(jnp.sum(u2 * wv))
        y = jnp.dot(x_ref[...], wcat_ref[...], preferred_element_type=jnp.float32)
        h = y[:, :D] / (sigma + 1e-12)
        h2_ref[0] = h[:, :D // 2]
        h2_ref[1] = h[:, D // 2:]
        a_ref[...] = y[:, D:2 * D]
        bb_ref[...] = y[:, 2 * D:] + b1_ref[...]

    return pl.pallas_call(
        body,
        grid=(N // R,),
        in_specs=[
            pl.BlockSpec((R, D), lambda i: (i, 0)),
            pl.BlockSpec((D, 3 * D), lambda i: (0, 0)),
            pl.BlockSpec((1, D), lambda i: (0, 0)),
            pl.BlockSpec((1, D), lambda i: (0, 0)),
        ],
        out_specs=[
            pl.BlockSpec((2, R, D // 2), lambda i: (0, i, 0)),
            pl.BlockSpec((R, D), lambda i: (i, 0)),
            pl.BlockSpec((R, D), lambda i: (i, 0)),
        ],
        out_shape=[
            jax.ShapeDtypeStruct((2, N, D // 2), jnp.float32),
            jax.ShapeDtypeStruct((N, D), jnp.float32),
            jax.ShapeDtypeStruct((N, D), jnp.float32),
        ],
    )(x, wcat, u2d, b1)


def _sc_gates(A, B, w2, b2v, src_p, dst_p, E, E_pad, D):
    epw = E_pad // NW
    n_chunks = epw // K

    def body(a_hbm, b_hbm, w2_hbm, b2_hbm, src_hbm, dst_hbm, gates_hbm,
             w2_v, b2_v, idx_s, idx_d, rows_a, rows_b, gates_v, sem_a, sem_b):
        cid = lax.axis_index("c")
        sid = lax.axis_index("s")
        wid = sid * NC + cid
        base = wid * epw
        pltpu.sync_copy(w2_hbm, w2_v)
        pltpu.sync_copy(b2_hbm, b2_v)
        b2 = b2_v[...]
        w2vecs = [w2_v[pl.ds(t * LANES, LANES)] for t in range(D // LANES)]
        iota = lax.iota(jnp.int32, LANES)

        def chunk_body(c, carry):
            cb = base + c * K
            pltpu.sync_copy(src_hbm.at[pl.ds(cb, K)], idx_s)
            pltpu.sync_copy(dst_hbm.at[pl.ds(cb, K)], idx_d)
            cpa = pltpu.async_copy(a_hbm.at[idx_s], rows_a, sem_a)
            cpb = pltpu.async_copy(b_hbm.at[idx_d], rows_b, sem_b)
            cpa.wait()
            cpb.wait()

            def group_body(g, carry2):
                gvec = jnp.zeros((LANES,), jnp.float32)
                for l in range(LANES):
                    e = g * LANES + l
                    acc0 = jnp.zeros((LANES,), jnp.float32)
                    acc1 = jnp.zeros((LANES,), jnp.float32)
                    for j in range(D // LANES):
                        sl = pl.ds(j * LANES, LANES)
                        t = jnp.maximum(rows_a[e, sl] + rows_b[e, sl], 0.0) * w2vecs[j]
                        if j % 2 == 0:
                            acc0 = acc0 + t
                        else:
                            acc1 = acc1 + t
                    s = jnp.sum(acc0 + acc1)
                    gvec = jnp.where(iota == l, s, gvec)
                gate = 1.0 / (1.0 + jnp.exp(-(gvec + b2)))
                gates_v[pl.ds(g * LANES, LANES)] = gate
                return carry2

            lax.fori_loop(0, K // LANES, group_body, 0)
            pltpu.sync_copy(gates_v, gates_hbm.at[pl.ds(cb, K)])
            return carry

        lax.fori_loop(0, n_chunks, chunk_body, 0)

    mesh = plsc.VectorSubcoreMesh(core_axis_name="c", subcore_axis_name="s",
                                  num_cores=NC, num_subcores=NS)
    return pl.kernel(
        body,
        out_type=jax.ShapeDtypeStruct((E_pad,), jnp.float32),
        mesh=mesh,
        compiler_params=pltpu.CompilerParams(use_tc_tiling_on_sc=False, needs_layout_passes=False),
        scratch_types=[
            pltpu.VMEM((D,), jnp.float32),
            pltpu.VMEM((LANES,), jnp.float32),
            pltpu.VMEM((K,), jnp.int32),
            pltpu.VMEM((K,), jnp.int32),
            pltpu.VMEM((K, D), jnp.float32),
            pltpu.VMEM((K, D), jnp.float32),
            pltpu.VMEM((K,), jnp.float32),
            pltpu.SemaphoreType.DMA,
            pltpu.SemaphoreType.DMA,
        ],
    )(A, B, w2, b2v, src_p, dst_p)


def _sc_agg(h_cat, gates, src_p, dst_p, N, E, E_pad, D):
    DH = D // 2
    ept = E_pad // NS
    n_chunks = ept // K
    rpt = N // NS

    def body(h_hbm, gates_hbm, src_hbm, dst_hbm, agg_hbm, cnt_hbm,
             idx_h, idx_d, gates_v, rows_h, ones_v, agg_s, cnt_s, sem_h):
        cid = lax.axis_index("c")
        sid = lax.axis_index("s")
        base = sid * ept
        row0 = sid * rpt
        cid_n = cid * N

        def zrow(e, carry):
            for j in range(DH // LANES):
                rows_h[e, pl.ds(j * LANES, LANES)] = jnp.zeros((LANES,), jnp.float32)
            ones_v[e, pl.ds(0, LANES)] = jnp.zeros((LANES,), jnp.float32)
            return carry

        lax.fori_loop(0, K, zrow, 0)
        off = 0
        while off < rpt:
            n = min(K, rpt - off)
            pltpu.sync_copy(rows_h.at[pl.ds(0, n)], agg_s.at[pl.ds(row0 + off, n)])
            pltpu.sync_copy(ones_v.at[pl.ds(0, n)], cnt_s.at[pl.ds(row0 + off, n)])
            off += n
        plsc.subcore_barrier()

        def chunk_body(c, carry):
            cb = base + c * K
            pltpu.sync_copy(src_hbm.at[pl.ds(cb, K)], idx_h)
            pltpu.sync_copy(dst_hbm.at[pl.ds(cb, K)], idx_d)
            pltpu.sync_copy(gates_hbm.at[pl.ds(cb, K)], gates_v)
            for g in range(K // LANES):
                sl = pl.ds(g * LANES, LANES)
                idx_h[sl] = idx_h[sl] + cid_n
            pltpu.async_copy(h_hbm.at[idx_h], rows_h, sem_h).wait()

            iota = lax.iota(jnp.int32, LANES)

            def group_body(g, carry2):
                gvec = gates_v[pl.ds(g * LANES, LANES)]
                valid = (cb + g * LANES + iota) < E
                gm = jnp.where(valid, gvec, 0.0)
                om = jnp.where(valid, 1.0, 0.0)
                for l in range(LANES):
                    e = g * LANES + l
                    gt = gm[l]
                    for j in range(DH // LANES):
                        sl = pl.ds(j * LANES, LANES)
                        rows_h[e, sl] = rows_h[e, sl] * gt
                    ones_v[e, pl.ds(0, LANES)] = jnp.zeros((LANES,), jnp.float32) + om[l]
                return carry2

            lax.fori_loop(0, K // LANES, group_body, 0)
            pltpu.sync_copy(rows_h, agg_s.at[idx_d], add=True)

            @pl.when(cid == 0)
            def _():
                pltpu.sync_copy(ones_v, cnt_s.at[idx_d], add=True)

            return carry

        lax.fori_loop(0, n_chunks, chunk_body, 0)
        plsc.subcore_barrier()
        pltpu.sync_copy(agg_s.at[pl.ds(row0, rpt)],
                        agg_hbm.at[pl.ds(cid_n + row0, rpt)])

        @pl.when(cid == 0)
        def _():
            pltpu.sync_copy(cnt_s.at[pl.ds(row0, rpt)],
                            cnt_hbm.at[pl.ds(row0, rpt)])

    mesh = plsc.VectorSubcoreMesh(core_axis_name="c", subcore_axis_name="s",
                                  num_cores=NC, num_subcores=NS)
    return pl.kernel(
        body,
        out_type=[
            jax.ShapeDtypeStruct((2 * N, DH), jnp.float32),
            jax.ShapeDtypeStruct((N, LANES), jnp.float32),
        ],
        mesh=mesh,
        compiler_params=pltpu.CompilerParams(use_tc_tiling_on_sc=False, needs_layout_passes=False),
        scratch_types=[
            pltpu.VMEM((K,), jnp.int32),
            pltpu.VMEM((K,), jnp.int32),
            pltpu.VMEM((K,), jnp.float32),
            pltpu.VMEM((K, DH), jnp.float32),
            pltpu.VMEM((K, LANES), jnp.float32),
            pltpu.VMEM_SHARED((N, DH), jnp.float32),
            pltpu.VMEM_SHARED((N, LANES), jnp.float32),
            pltpu.SemaphoreType.DMA,
        ],
    )(h_cat, gates, src_p, dst_p)


def _tc_post(x, agg2, cnt, wc, bc, g2d, be2d, N, D, R):
    DH = D // 2

    def body(x_ref, agg_ref, cnt_ref, wc_ref, bc_ref, g_ref, be_ref, o_ref):
        c = cnt_ref[:, 0:1]
        c = jnp.where(c == 0.0, 1.0, c)
        at = agg_ref[0] / c
        ab = agg_ref[1] / c
        o = (jnp.dot(x_ref[...], wc_ref[:D, :], preferred_element_type=jnp.float32)
             + jnp.dot(at, wc_ref[D:D + DH, :], preferred_element_type=jnp.float32)
             + jnp.dot(ab, wc_ref[D + DH:, :], preferred_element_type=jnp.float32))
        o = jnp.maximum(o + bc_ref[...], 0.0)
        mean = jnp.mean(o, axis=1, keepdims=True)
        var = jnp.mean((o - mean) ** 2, axis=1, keepdims=True)
        o = (o - mean) / jnp.sqrt(var + 1e-5) * g_ref[...] + be_ref[...]
        o_ref[...] = o

    return pl.pallas_call(
        body,
        grid=(N // R,),
        in_specs=[
            pl.BlockSpec((R, D), lambda i: (i, 0)),
            pl.BlockSpec((2, R, DH), lambda i: (0, i, 0)),
            pl.BlockSpec((R, LANES), lambda i: (i, 0)),
            pl.BlockSpec((2 * D, D), lambda i: (0, 0)),
            pl.BlockSpec((1, D), lambda i: (0, 0)),
            pl.BlockSpec((1, D), lambda i: (0, 0)),
            pl.BlockSpec((1, D), lambda i: (0, 0)),
        ],
        out_specs=pl.BlockSpec((R, D), lambda i: (i, 0)),
        out_shape=jax.ShapeDtypeStruct((N, D), jnp.float32),
    )(x, agg2, cnt, wc, bc, g2d, be2d)


def kernel(x, W, u, W_lp1, b_lp1, W_lp2, b_lp2, W_comb, b_comb, gamma, beta,
           edge_index):
    N, D = x.shape
    E = edge_index.shape[1]
    R = 1000

    chunk_span = NW * K
    E_pad = ((E + chunk_span - 1) // chunk_span) * chunk_span

    wcat = jnp.concatenate([W, W_lp1[:D], W_lp1[D:]], axis=1)
    u2d = u.reshape(1, D)
    b1 = b_lp1.reshape(1, D)
    w2 = W_lp2[:, 0]
    b2v = jnp.full((LANES,), b_lp2[0], dtype=jnp.float32)

    src = edge_index[0]
    dst = edge_index[1]
    pad = E_pad - E
    if pad:
        zpad = jnp.zeros((pad,), dtype=jnp.int32)
        src_p = jnp.concatenate([src, zpad])
        dst_p = jnp.concatenate([dst, zpad])
    else:
        src_p, dst_p = src, dst

    h2, A, B = _tc_pre(x, wcat, u2d, b1, N, D, R)
    h_cat = h2.reshape(2 * N, D // 2)
    gates = _sc_gates(A, B, w2, b2v, src_p, dst_p, E, E_pad, D)
    agg_cat, cnt = _sc_agg(h_cat, gates, src_p, dst_p, N, E, E_pad, D)
    agg2 = agg_cat.reshape(2, N, D // 2)
    out = _tc_post(x, agg2, cnt, W_comb, b_comb.reshape(1, D), gamma.reshape(1, D),
                   beta.reshape(1, D), N, D, R)
    return out


# speedup vs baseline: 1.4447x; 1.4447x over previous
import jax
import jax.numpy as jnp
from jax import lax
from jax.experimental import pallas as pl
from jax.experimental.pallas import tpu as pltpu
from jax.experimental.pallas import tpu_sc as plsc

NC = 2
NS = 16
NW = NC * NS
LANES = 16
KG = 64
K = 128


def _tc_pre(x, wcat, u2d, b1, N, D, R):
    def body(x_ref, wcat_ref, u_ref, b1_ref, h2_ref, a_ref, bb_ref):
        W = wcat_ref[:, :D]
        u = u_ref[...]
        v = jnp.dot(u, W, preferred_element_type=jnp.float32)
        v = v / (jnp.sqrt(jnp.sum(v * v)) + 1e-12)
        wv = lax.dot_general(v, W, (((1,), (1,)), ((), ())),
                             preferred_element_type=jnp.float32)
        u2 = wv / (jnp.sqrt(jnp.sum(wv * wv)) + 1e-12)
        sigma = jnp.abs(jnp.sum(u2 * wv))
        y = jnp.dot(x_ref[...], wcat_ref[...], preferred_element_type=jnp.float32)
        h = y[:, :D] / (sigma + 1e-12)
        h2_ref[0] = h[:, :D // 2]
        h2_ref[1] = h[:, D // 2:]
        a_ref[...] = y[:, D:2 * D]
        bb_ref[...] = y[:, 2 * D:] + b1_ref[...]

    return pl.pallas_call(
        body,
        grid=(N // R,),
        in_specs=[
            pl.BlockSpec((R, D), lambda i: (i, 0)),
            pl.BlockSpec((D, 3 * D), lambda i: (0, 0)),
            pl.BlockSpec((1, D), lambda i: (0, 0)),
            pl.BlockSpec((1, D), lambda i: (0, 0)),
        ],
        out_specs=[
            pl.BlockSpec((2, R, D // 2), lambda i: (0, i, 0)),
            pl.BlockSpec((R, D), lambda i: (i, 0)),
            pl.BlockSpec((R, D), lambda i: (i, 0)),
        ],
        out_shape=[
            jax.ShapeDtypeStruct((2, N, D // 2), jnp.float32),
            jax.ShapeDtypeStruct((N, D), jnp.float32),
            jax.ShapeDtypeStruct((N, D), jnp.float32),
        ],
    )(x, wcat, u2d, b1)


def _sc_gates(A, B, w2, b2v, src_p, dst_p, E, E_pad, D):
    epw = E_pad // NW
    n_chunks = epw // KG
    n2 = n_chunks // 2

    def body(a_hbm, b_hbm, w2_hbm, b2_hbm, src_hbm, dst_hbm, gates_hbm,
             w2_v, b2_v, idx_s_all, idx_d_all, gates_all,
             rows_aa, rows_ba, rows_ab, rows_bb,
             sem_aa, sem_ba, sem_ab, sem_bb):
        cid = lax.axis_index("c")
        sid = lax.axis_index("s")
        wid = sid * NC + cid
        base = wid * epw
        pltpu.sync_copy(w2_hbm, w2_v)
        pltpu.sync_copy(b2_hbm, b2_v)
        pltpu.sync_copy(src_hbm.at[pl.ds(base, epw)], idx_s_all)
        pltpu.sync_copy(dst_hbm.at[pl.ds(base, epw)], idx_d_all)
        b2 = b2_v[...]
        w2vecs = [w2_v[pl.ds(t * LANES, LANES)] for t in range(D // LANES)]
        iota = lax.iota(jnp.int32, LANES)

        def start_gather(off, ra, rb, sa, sb):
            isl = idx_s_all.at[pl.ds(off, KG)]
            idl = idx_d_all.at[pl.ds(off, KG)]
            pltpu.make_async_copy(a_hbm.at[isl], ra, sa).start()
            pltpu.make_async_copy(b_hbm.at[idl], rb, sb).start()

        def wait_gather(off, ra, rb, sa, sb):
            isl = idx_s_all.at[pl.ds(off, KG)]
            idl = idx_d_all.at[pl.ds(off, KG)]
            pltpu.make_async_copy(a_hbm.at[isl], ra, sa).wait()
            pltpu.make_async_copy(b_hbm.at[idl], rb, sb).wait()

        def compute_chunk(ra, rb, off):
            def group_body(g, carry):
                gvec = jnp.zeros((LANES,), jnp.float32)
                for l in range(LANES):
                    e = g * LANES + l
                    acc0 = jnp.zeros((LANES,), jnp.float32)
                    acc1 = jnp.zeros((LANES,), jnp.float32)
                    for j in range(D // LANES):
                        sl = pl.ds(j * LANES, LANES)
                        t = jnp.maximum(ra[e, sl] + rb[e, sl], 0.0) * w2vecs[j]
                        if j % 2 == 0:
                            acc0 = acc0 + t
                        else:
                            acc1 = acc1 + t
                    s = jnp.sum(acc0 + acc1)
                    gvec = jnp.where(iota == l, s, gvec)
                gate = 1.0 / (1.0 + jnp.exp(-(gvec + b2)))
                gates_all[pl.ds(off + g * LANES, LANES)] = gate
                return carry

            lax.fori_loop(0, KG // LANES, group_body, 0)

        start_gather(0, rows_aa, rows_ba, sem_aa, sem_ba)

        def body2(c2, carry):
            off = 2 * c2 * KG
            start_gather(off + KG, rows_ab, rows_bb, sem_ab, sem_bb)
            wait_gather(off, rows_aa, rows_ba, sem_aa, sem_ba)
            compute_chunk(rows_aa, rows_ba, off)

            @pl.when(c2 + 1 < n2)
            def _():
                start_gather(off + 2 * KG, rows_aa, rows_ba, sem_aa, sem_ba)

            wait_gather(off + KG, rows_ab, rows_bb, sem_ab, sem_bb)
            compute_chunk(rows_ab, rows_bb, off + KG)
            return carry

        lax.fori_loop(0, n2, body2, 0)
        pltpu.sync_copy(gates_all, gates_hbm.at[pl.ds(base, epw)])

    mesh = plsc.VectorSubcoreMesh(core_axis_name="c", subcore_axis_name="s",
                                  num_cores=NC, num_subcores=NS)
    return pl.kernel(
        body,
        out_type=jax.ShapeDtypeStruct((E_pad,), jnp.float32),
        mesh=mesh,
        compiler_params=pltpu.CompilerParams(use_tc_tiling_on_sc=False, needs_layout_passes=False),
        scratch_types=[
            pltpu.VMEM((D,), jnp.float32),
            pltpu.VMEM((LANES,), jnp.float32),
            pltpu.VMEM((E_pad // NW,), jnp.int32),
            pltpu.VMEM((E_pad // NW,), jnp.int32),
            pltpu.VMEM((E_pad // NW,), jnp.float32),
            pltpu.VMEM((KG, D), jnp.float32),
            pltpu.VMEM((KG, D), jnp.float32),
            pltpu.VMEM((KG, D), jnp.float32),
            pltpu.VMEM((KG, D), jnp.float32),
            pltpu.SemaphoreType.DMA,
            pltpu.SemaphoreType.DMA,
            pltpu.SemaphoreType.DMA,
            pltpu.SemaphoreType.DMA,
        ],
    )(A, B, w2, b2v, src_p, dst_p)


def _sc_agg(h_cat, gates, src_p, dst_p, N, E, E_pad, D):
    DH = D // 2
    ept = E_pad // NS
    n_chunks = ept // K
    n2 = n_chunks // 2
    rpt = N // NS

    def body(h_hbm, gates_hbm, src_hbm, dst_hbm, agg_hbm, cnt_hbm,
             idx_ha, idx_da, gates_va, idx_hb, idx_db, gates_vb,
             rows_ha, rows_hb, ones_v,
             agg_s, cnt_s, sem_ha, sem_hb):
        cid = lax.axis_index("c")
        sid = lax.axis_index("s")
        base = sid * ept
        row0 = sid * rpt
        cid_n = cid * N

        def zrow(e, carry):
            for j in range(DH // LANES):
                rows_ha[e, pl.ds(j * LANES, LANES)] = jnp.zeros((LANES,), jnp.float32)
            ones_v[e, pl.ds(0, LANES)] = jnp.zeros((LANES,), jnp.float32)
            return carry

        lax.fori_loop(0, K, zrow, 0)
        off = 0
        while off < rpt:
            n = min(K, rpt - off)
            pltpu.sync_copy(rows_ha.at[pl.ds(0, n)], agg_s.at[pl.ds(row0 + off, n)])
            pltpu.sync_copy(ones_v.at[pl.ds(0, n)], cnt_s.at[pl.ds(row0 + off, n)])
            off += n
        plsc.subcore_barrier()

        iota = lax.iota(jnp.int32, LANES)

        def load_idx_start(off, ih, idd, gv, rh, sh):
            cb = base + off
            pltpu.sync_copy(src_hbm.at[pl.ds(cb, K)], ih)
            pltpu.sync_copy(dst_hbm.at[pl.ds(cb, K)], idd)
            pltpu.sync_copy(gates_hbm.at[pl.ds(cb, K)], gv)
            for g in range(K // LANES):
                sl = pl.ds(g * LANES, LANES)
                ih[sl] = ih[sl] + cid_n
            pltpu.make_async_copy(h_hbm.at[ih], rh, sh).start()

        def process_chunk(off, ih, idd, gv, rh, sh):
            pltpu.make_async_copy(h_hbm.at[ih], rh, sh).wait()

            def group_body(g, carry2):
                gvec = gv[pl.ds(g * LANES, LANES)]
                valid = (base + off + g * LANES + iota) < E
                gm = jnp.where(valid, gvec, 0.0)
                om = jnp.where(valid, 1.0, 0.0)
                for l in range(LANES):
                    e = g * LANES + l
                    gt = gm[l]
                    for j in range(DH // LANES):
                        sl = pl.ds(j * LANES, LANES)
                        rh[e, sl] = rh[e, sl] * gt
                    ones_v[e, pl.ds(0, LANES)] = jnp.zeros((LANES,), jnp.float32) + om[l]
                return carry2

            lax.fori_loop(0, K // LANES, group_body, 0)
            pltpu.sync_copy(rh, agg_s.at[idd], add=True)

            @pl.when(cid == 0)
            def _():
                pltpu.sync_copy(ones_v, cnt_s.at[idd], add=True)

        load_idx_start(0, idx_ha, idx_da, gates_va, rows_ha, sem_ha)

        def body2(c2, carry):
            off = 2 * c2 * K
            load_idx_start(off + K, idx_hb, idx_db, gates_vb, rows_hb, sem_hb)
            process_chunk(off, idx_ha, idx_da, gates_va, rows_ha, sem_ha)

            @pl.when(c2 + 1 < n2)
            def _():
                load_idx_start(off + 2 * K, idx_ha, idx_da, gates_va,
                               rows_ha, sem_ha)

            process_chunk(off + K, idx_hb, idx_db, gates_vb, rows_hb, sem_hb)
            return carry

        lax.fori_loop(0, n2, body2, 0)
        plsc.subcore_barrier()
        pltpu.sync_copy(agg_s.at[pl.ds(row0, rpt)],
                        agg_hbm.at[pl.ds(cid_n + row0, rpt)])

        @pl.when(cid == 0)
        def _():
            pltpu.sync_copy(cnt_s.at[pl.ds(row0, rpt)],
                            cnt_hbm.at[pl.ds(row0, rpt)])

    mesh = plsc.VectorSubcoreMesh(core_axis_name="c", subcore_axis_name="s",
                                  num_cores=NC, num_subcores=NS)
    return pl.kernel(
        body,
        out_type=[
            jax.ShapeDtypeStruct((2 * N, DH), jnp.float32),
            jax.ShapeDtypeStruct((N, LANES), jnp.float32),
        ],
        mesh=mesh,
        compiler_params=pltpu.CompilerParams(use_tc_tiling_on_sc=False, needs_layout_passes=False),
        scratch_types=[
            pltpu.VMEM((K,), jnp.int32),
            pltpu.VMEM((K,), jnp.int32),
            pltpu.VMEM((K,), jnp.float32),
            pltpu.VMEM((K,), jnp.int32),
            pltpu.VMEM((K,), jnp.int32),
            pltpu.VMEM((K,), jnp.float32),
            pltpu.VMEM((K, DH), jnp.float32),
            pltpu.VMEM((K, DH), jnp.float32),
            pltpu.VMEM((K, LANES), jnp.float32),
            pltpu.VMEM_SHARED((N, DH), jnp.float32),
            pltpu.VMEM_SHARED((N, LANES), jnp.float32),
            pltpu.SemaphoreType.DMA,
            pltpu.SemaphoreType.DMA,
        ],
    )(h_cat, gates, src_p, dst_p)


def _tc_post(x, agg2, cnt, wc, bc, g2d, be2d, N, D, R):
    DH = D // 2

    def body(x_ref, agg_ref, cnt_ref, wc_ref, bc_ref, g_ref, be_ref, o_ref):
        c = cnt_ref[:, 0:1]
        c = jnp.where(c == 0.0, 1.0, c)
        at = agg_ref[0] / c
        ab = agg_ref[1] / c
        o = (jnp.dot(x_ref[...], wc_ref[:D, :], preferred_element_type=jnp.float32)
             + jnp.dot(at, wc_ref[D:D + DH, :], preferred_element_type=jnp.float32)
             + jnp.dot(ab, wc_ref[D + DH:, :], preferred_element_type=jnp.float32))
        o = jnp.maximum(o + bc_ref[...], 0.0)
        mean = jnp.mean(o, axis=1, keepdims=True)
        var = jnp.mean((o - mean) ** 2, axis=1, keepdims=True)
        o = (o - mean) / jnp.sqrt(var + 1e-5) * g_ref[...] + be_ref[...]
        o_ref[...] = o

    return pl.pallas_call(
        body,
        grid=(N // R,),
        in_specs=[
            pl.BlockSpec((R, D), lambda i: (i, 0)),
            pl.BlockSpec((2, R, DH), lambda i: (0, i, 0)),
            pl.BlockSpec((R, LANES), lambda i: (i, 0)),
            pl.BlockSpec((2 * D, D), lambda i: (0, 0)),
            pl.BlockSpec((1, D), lambda i: (0, 0)),
            pl.BlockSpec((1, D), lambda i: (0, 0)),
            pl.BlockSpec((1, D), lambda i: (0, 0)),
        ],
        out_specs=pl.BlockSpec((R, D), lambda i: (i, 0)),
        out_shape=jax.ShapeDtypeStruct((N, D), jnp.float32),
    )(x, agg2, cnt, wc, bc, g2d, be2d)


def kernel(x, W, u, W_lp1, b_lp1, W_lp2, b_lp2, W_comb, b_comb, gamma, beta,
           edge_index):
    N, D = x.shape
    E = edge_index.shape[1]
    R = 1000

    chunk_span = NW * KG * 2
    E_pad = ((E + chunk_span - 1) // chunk_span) * chunk_span

    wcat = jnp.concatenate([W, W_lp1[:D], W_lp1[D:]], axis=1)
    u2d = u.reshape(1, D)
    b1 = b_lp1.reshape(1, D)
    w2 = W_lp2[:, 0]
    b2v = jnp.full((LANES,), b_lp2[0], dtype=jnp.float32)

    src = edge_index[0]
    dst = edge_index[1]
    pad = E_pad - E
    if pad:
        zpad = jnp.zeros((pad,), dtype=jnp.int32)
        src_p = jnp.concatenate([src, zpad])
        dst_p = jnp.concatenate([dst, zpad])
    else:
        src_p, dst_p = src, dst

    h2, A, B = _tc_pre(x, wcat, u2d, b1, N, D, R)
    h_cat = h2.reshape(2 * N, D // 2)
    gates = _sc_gates(A, B, w2, b2v, src_p, dst_p, E, E_pad, D)
    agg_cat, cnt = _sc_agg(h_cat, gates, src_p, dst_p, N, E, E_pad, D)
    agg2 = agg_cat.reshape(2, N, D // 2)
    out = _tc_post(x, agg2, cnt, W_comb, b_comb.reshape(1, D), gamma.reshape(1, D),
                   beta.reshape(1, D), N, D, R)
    return out


# speedup vs baseline: 1.5739x; 1.0894x over previous
import jax
import jax.numpy as jnp
from jax import lax
from jax.experimental import pallas as pl
from jax.experimental.pallas import tpu as pltpu
from jax.experimental.pallas import tpu_sc as plsc

NC = 2
NS = 16
NW = NC * NS
LANES = 16
KG = 64
K = 128


def _tc_pre(x, wcat, u2d, b1, N, D, R):
    def body(x_ref, wcat_ref, u_ref, b1_ref, h2_ref, a_ref, bb_ref):
        W = wcat_ref[:, :D]
        u = u_ref[...]
        v = jnp.dot(u, W, preferred_element_type=jnp.float32)
        v = v / (jnp.sqrt(jnp.sum(v * v)) + 1e-12)
        wv = lax.dot_general(v, W, (((1,), (1,)), ((), ())),
                             preferred_element_type=jnp.float32)
        u2 = wv / (jnp.sqrt(jnp.sum(wv * wv)) + 1e-12)
        sigma = jnp.abs(jnp.sum(u2 * wv))
        y = jnp.dot(x_ref[...], wcat_ref[...], preferred_element_type=jnp.float32)
        h = y[:, :D] / (sigma + 1e-12)
        h2_ref[0] = h[:, :D // 2]
        h2_ref[1] = h[:, D // 2:]
        a_ref[...] = y[:, D:2 * D].astype(jnp.bfloat16)
        bb_ref[...] = (y[:, 2 * D:] + b1_ref[...]).astype(jnp.bfloat16)

    return pl.pallas_call(
        body,
        grid=(N // R,),
        in_specs=[
            pl.BlockSpec((R, D), lambda i: (i, 0)),
            pl.BlockSpec((D, 3 * D), lambda i: (0, 0)),
            pl.BlockSpec((1, D), lambda i: (0, 0)),
            pl.BlockSpec((1, D), lambda i: (0, 0)),
        ],
        out_specs=[
            pl.BlockSpec((2, R, D // 2), lambda i: (0, i, 0)),
            pl.BlockSpec((R, D), lambda i: (i, 0)),
            pl.BlockSpec((R, D), lambda i: (i, 0)),
        ],
        out_shape=[
            jax.ShapeDtypeStruct((2, N, D // 2), jnp.float32),
            jax.ShapeDtypeStruct((N, D), jnp.bfloat16),
            jax.ShapeDtypeStruct((N, D), jnp.bfloat16),
        ],
    )(x, wcat, u2d, b1)


def _sc_gates(A, B, w2, b2v, src_p, dst_p, E, E_pad, D):
    epw = E_pad // NW
    n_chunks = epw // KG
    n2 = n_chunks // 2

    def body(a_hbm, b_hbm, w2_hbm, b2_hbm, src_hbm, dst_hbm, gates_hbm,
             w2_v, b2_v, idx_s_all, idx_d_all, gates_all,
             rows_aa, rows_ba, rows_ab, rows_bb,
             sem_aa, sem_ba, sem_ab, sem_bb):
        cid = lax.axis_index("c")
        sid = lax.axis_index("s")
        wid = sid * NC + cid
        base = wid * epw
        pltpu.sync_copy(w2_hbm, w2_v)
        pltpu.sync_copy(b2_hbm, b2_v)
        pltpu.sync_copy(src_hbm.at[pl.ds(base, epw)], idx_s_all)
        pltpu.sync_copy(dst_hbm.at[pl.ds(base, epw)], idx_d_all)
        b2 = b2_v[...]
        BL = 2 * LANES
        w2vecs = [w2_v[pl.ds(t * BL, BL)] for t in range(D // BL)]
        iota = lax.iota(jnp.int32, LANES)

        def start_gather(off, ra, rb, sa, sb):
            isl = idx_s_all.at[pl.ds(off, KG)]
            idl = idx_d_all.at[pl.ds(off, KG)]
            pltpu.make_async_copy(a_hbm.at[isl], ra, sa).start()
            pltpu.make_async_copy(b_hbm.at[idl], rb, sb).start()

        def wait_gather(off, ra, rb, sa, sb):
            isl = idx_s_all.at[pl.ds(off, KG)]
            idl = idx_d_all.at[pl.ds(off, KG)]
            pltpu.make_async_copy(a_hbm.at[isl], ra, sa).wait()
            pltpu.make_async_copy(b_hbm.at[idl], rb, sb).wait()

        def compute_chunk(ra, rb, off):
            def group_body(g, carry):
                gvec = jnp.zeros((LANES,), jnp.float32)
                zero_b = jnp.zeros((BL,), jnp.bfloat16)
                for l in range(LANES):
                    e = g * LANES + l
                    acc0 = jnp.zeros((LANES,), jnp.float32)
                    acc1 = jnp.zeros((LANES,), jnp.float32)
                    for j in range(D // BL):
                        sl = pl.ds(j * BL, BL)
                        t = jnp.maximum(ra[e, sl] + rb[e, sl], zero_b) * w2vecs[j]
                        te, to = plsc.unpack(t, format=plsc.PackFormat.INTERLEAVED,
                                             preferred_element_type=jnp.float32)
                        acc0 = acc0 + te
                        acc1 = acc1 + to
                    s = jnp.sum(acc0 + acc1)
                    gvec = jnp.where(iota == l, s, gvec)
                gate = 1.0 / (1.0 + jnp.exp(-(gvec + b2)))
                gates_all[pl.ds(off + g * LANES, LANES)] = gate
                return carry

            lax.fori_loop(0, KG // LANES, group_body, 0)

        start_gather(0, rows_aa, rows_ba, sem_aa, sem_ba)

        def body2(c2, carry):
            off = 2 * c2 * KG
            start_gather(off + KG, rows_ab, rows_bb, sem_ab, sem_bb)
            wait_gather(off, rows_aa, rows_ba, sem_aa, sem_ba)
            compute_chunk(rows_aa, rows_ba, off)

            @pl.when(c2 + 1 < n2)
            def _():
                start_gather(off + 2 * KG, rows_aa, rows_ba, sem_aa, sem_ba)

            wait_gather(off + KG, rows_ab, rows_bb, sem_ab, sem_bb)
            compute_chunk(rows_ab, rows_bb, off + KG)
            return carry

        lax.fori_loop(0, n2, body2, 0)
        pltpu.sync_copy(gates_all, gates_hbm.at[pl.ds(base, epw)])

    mesh = plsc.VectorSubcoreMesh(core_axis_name="c", subcore_axis_name="s",
                                  num_cores=NC, num_subcores=NS)
    return pl.kernel(
        body,
        out_type=jax.ShapeDtypeStruct((E_pad,), jnp.float32),
        mesh=mesh,
        compiler_params=pltpu.CompilerParams(use_tc_tiling_on_sc=False, needs_layout_passes=False),
        scratch_types=[
            pltpu.VMEM((D,), jnp.bfloat16),
            pltpu.VMEM((LANES,), jnp.float32),
            pltpu.VMEM((E_pad // NW,), jnp.int32),
            pltpu.VMEM((E_pad // NW,), jnp.int32),
            pltpu.VMEM((E_pad // NW,), jnp.float32),
            pltpu.VMEM((KG, D), jnp.bfloat16),
            pltpu.VMEM((KG, D), jnp.bfloat16),
            pltpu.VMEM((KG, D), jnp.bfloat16),
            pltpu.VMEM((KG, D), jnp.bfloat16),
            pltpu.SemaphoreType.DMA,
            pltpu.SemaphoreType.DMA,
            pltpu.SemaphoreType.DMA,
            pltpu.SemaphoreType.DMA,
        ],
    )(A, B, w2, b2v, src_p, dst_p)


def _sc_agg(h_cat, gates, src_p, dst_p, N, E, E_pad, D):
    DH = D // 2
    ept = E_pad // NS
    n_chunks = ept // K
    n2 = n_chunks // 2
    rpt = N // NS

    def body(h_hbm, gates_hbm, src_hbm, dst_hbm, agg_hbm, cnt_hbm,
             idx_ha, idx_da, gates_va, idx_hb, idx_db, gates_vb,
             rows_ha, rows_hb, ones_v,
             agg_s, cnt_s, sem_ha, sem_hb):
        cid = lax.axis_index("c")
        sid = lax.axis_index("s")
        base = sid * ept
        row0 = sid * rpt
        cid_n = cid * N

        def zrow(e, carry):
            for j in range(DH // LANES):
                rows_ha[e, pl.ds(j * LANES, LANES)] = jnp.zeros((LANES,), jnp.float32)
            ones_v[e, pl.ds(0, LANES)] = jnp.zeros((LANES,), jnp.float32)
            return carry

        lax.fori_loop(0, K, zrow, 0)
        off = 0
        while off < rpt:
            n = min(K, rpt - off)
            pltpu.sync_copy(rows_ha.at[pl.ds(0, n)], agg_s.at[pl.ds(row0 + off, n)])
            pltpu.sync_copy(ones_v.at[pl.ds(0, n)], cnt_s.at[pl.ds(row0 + off, n)])
            off += n
        plsc.subcore_barrier()

        iota = lax.iota(jnp.int32, LANES)

        def load_idx_start(off, ih, idd, gv, rh, sh):
            cb = base + off
            pltpu.sync_copy(src_hbm.at[pl.ds(cb, K)], ih)
            pltpu.sync_copy(dst_hbm.at[pl.ds(cb, K)], idd)
            pltpu.sync_copy(gates_hbm.at[pl.ds(cb, K)], gv)
            for g in range(K // LANES):
                sl = pl.ds(g * LANES, LANES)
                ih[sl] = ih[sl] + cid_n
            pltpu.make_async_copy(h_hbm.at[ih], rh, sh).start()

        def process_chunk(off, ih, idd, gv, rh, sh):
            pltpu.make_async_copy(h_hbm.at[ih], rh, sh).wait()

            def group_body(g, carry2):
                gvec = gv[pl.ds(g * LANES, LANES)]
                valid = (base + off + g * LANES + iota) < E
                gm = jnp.where(valid, gvec, 0.0)
                om = jnp.where(valid, 1.0, 0.0)
                for l in range(LANES):
                    e = g * LANES + l
                    gt = gm[l]
                    for j in range(DH // LANES):
                        sl = pl.ds(j * LANES, LANES)
                        rh[e, sl] = rh[e, sl] * gt
                    ones_v[e, pl.ds(0, LANES)] = jnp.zeros((LANES,), jnp.float32) + om[l]
                return carry2

            lax.fori_loop(0, K // LANES, group_body, 0)
            pltpu.sync_copy(rh, agg_s.at[idd], add=True)

            @pl.when(cid == 0)
            def _():
                pltpu.sync_copy(ones_v, cnt_s.at[idd], add=True)

        load_idx_start(0, idx_ha, idx_da, gates_va, rows_ha, sem_ha)

        def body2(c2, carry):
            off = 2 * c2 * K
            load_idx_start(off + K, idx_hb, idx_db, gates_vb, rows_hb, sem_hb)
            process_chunk(off, idx_ha, idx_da, gates_va, rows_ha, sem_ha)

            @pl.when(c2 + 1 < n2)
            def _():
                load_idx_start(off + 2 * K, idx_ha, idx_da, gates_va,
                               rows_ha, sem_ha)

            process_chunk(off + K, idx_hb, idx_db, gates_vb, rows_hb, sem_hb)
            return carry

        lax.fori_loop(0, n2, body2, 0)
        plsc.subcore_barrier()
        pltpu.sync_copy(agg_s.at[pl.ds(row0, rpt)],
                        agg_hbm.at[pl.ds(cid_n + row0, rpt)])

        @pl.when(cid == 0)
        def _():
            pltpu.sync_copy(cnt_s.at[pl.ds(row0, rpt)],
                            cnt_hbm.at[pl.ds(row0, rpt)])

    mesh = plsc.VectorSubcoreMesh(core_axis_name="c", subcore_axis_name="s",
                                  num_cores=NC, num_subcores=NS)
    return pl.kernel(
        body,
        out_type=[
            jax.ShapeDtypeStruct((2 * N, DH), jnp.float32),
            jax.ShapeDtypeStruct((N, LANES), jnp.float32),
        ],
        mesh=mesh,
        compiler_params=pltpu.CompilerParams(use_tc_tiling_on_sc=False, needs_layout_passes=False),
        scratch_types=[
            pltpu.VMEM((K,), jnp.int32),
            pltpu.VMEM((K,), jnp.int32),
            pltpu.VMEM((K,), jnp.float32),
            pltpu.VMEM((K,), jnp.int32),
            pltpu.VMEM((K,), jnp.int32),
            pltpu.VMEM((K,), jnp.float32),
            pltpu.VMEM((K, DH), jnp.float32),
            pltpu.VMEM((K, DH), jnp.float32),
            pltpu.VMEM((K, LANES), jnp.float32),
            pltpu.VMEM_SHARED((N, DH), jnp.float32),
            pltpu.VMEM_SHARED((N, LANES), jnp.float32),
            pltpu.SemaphoreType.DMA,
            pltpu.SemaphoreType.DMA,
        ],
    )(h_cat, gates, src_p, dst_p)


def _tc_post(x, agg2, cnt, wc, bc, g2d, be2d, N, D, R):
    DH = D // 2

    def body(x_ref, agg_ref, cnt_ref, wc_ref, bc_ref, g_ref, be_ref, o_ref):
        c = cnt_ref[:, 0:1]
        c = jnp.where(c == 0.0, 1.0, c)
        at = agg_ref[0] / c
        ab = agg_ref[1] / c
        o = (jnp.dot(x_ref[...], wc_ref[:D, :], preferred_element_type=jnp.float32)
             + jnp.dot(at, wc_ref[D:D + DH, :], preferred_element_type=jnp.float32)
             + jnp.dot(ab, wc_ref[D + DH:, :], preferred_element_type=jnp.float32))
        o = jnp.maximum(o + bc_ref[...], 0.0)
        mean = jnp.mean(o, axis=1, keepdims=True)
        var = jnp.mean((o - mean) ** 2, axis=1, keepdims=True)
        o = (o - mean) / jnp.sqrt(var + 1e-5) * g_ref[...] + be_ref[...]
        o_ref[...] = o

    return pl.pallas_call(
        body,
        grid=(N // R,),
        in_specs=[
            pl.BlockSpec((R, D), lambda i: (i, 0)),
            pl.BlockSpec((2, R, DH), lambda i: (0, i, 0)),
            pl.BlockSpec((R, LANES), lambda i: (i, 0)),
            pl.BlockSpec((2 * D, D), lambda i: (0, 0)),
            pl.BlockSpec((1, D), lambda i: (0, 0)),
            pl.BlockSpec((1, D), lambda i: (0, 0)),
            pl.BlockSpec((1, D), lambda i: (0, 0)),
        ],
        out_specs=pl.BlockSpec((R, D), lambda i: (i, 0)),
        out_shape=jax.ShapeDtypeStruct((N, D), jnp.float32),
    )(x, agg2, cnt, wc, bc, g2d, be2d)


def kernel(x, W, u, W_lp1, b_lp1, W_lp2, b_lp2, W_comb, b_comb, gamma, beta,
           edge_index):
    N, D = x.shape
    E = edge_index.shape[1]
    R = 1000

    chunk_span = NW * KG * 2
    E_pad = ((E + chunk_span - 1) // chunk_span) * chunk_span

    wcat = jnp.concatenate([W, W_lp1[:D], W_lp1[D:]], axis=1)
    u2d = u.reshape(1, D)
    b1 = b_lp1.reshape(1, D)
    w2 = W_lp2[:, 0].astype(jnp.bfloat16)
    b2v = jnp.full((LANES,), b_lp2[0], dtype=jnp.float32)

    src = edge_index[0]
    dst = edge_index[1]
    pad = E_pad - E
    if pad:
        zpad = jnp.zeros((pad,), dtype=jnp.int32)
        src_p = jnp.concatenate([src, zpad])
        dst_p = jnp.concatenate([dst, zpad])
    else:
        src_p, dst_p = src, dst

    h2, A, B = _tc_pre(x, wcat, u2d, b1, N, D, R)
    h_cat = h2.reshape(2 * N, D // 2)
    gates = _sc_gates(A, B, w2, b2v, src_p, dst_p, E, E_pad, D)
    agg_cat, cnt = _sc_agg(h_cat, gates, src_p, dst_p, N, E, E_pad, D)
    agg2 = agg_cat.reshape(2, N, D // 2)
    out = _tc_post(x, agg2, cnt, W_comb, b_comb.reshape(1, D), gamma.reshape(1, D),
                   beta.reshape(1, D), N, D, R)
    return out
